# baseline (speedup 1.0000x reference)
import functools

import jax
import jax.numpy as jnp
from jax import lax
from jax.experimental import pallas as pl
from jax.experimental.pallas import tpu as pltpu
from jax.experimental.pallas import tpu_sc as plsc

SIZE = 100000
INPUT_SIZE = 128
CH = 160
NCHUNK = SIZE // CH
NW = 32
TMAX = (NCHUNK + NW - 1) // NW
G = CH // 16

_mesh = plsc.VectorSubcoreMesh(
    core_axis_name="c", subcore_axis_name="s", num_cores=2, num_subcores=16)


@functools.partial(
    pl.kernel,
    out_type=[
        jax.ShapeDtypeStruct((SIZE,), jnp.float32),
        jax.ShapeDtypeStruct((NW, 16), jnp.int32),
    ],
    mesh=_mesh,
    scratch_types=[
        pltpu.VMEM((CH, INPUT_SIZE), jnp.float32),
        pltpu.VMEM((CH, INPUT_SIZE), jnp.float32),
        pltpu.VMEM((CH,), jnp.float32),
        pltpu.VMEM((1, INPUT_SIZE), jnp.float32),
        pltpu.VMEM((16,), jnp.int32),
        pltpu.SemaphoreType.DMA,
        pltpu.SemaphoreType.DMA,
    ],
    compiler_params=pltpu.CompilerParams(needs_layout_passes=False),
)
def _sc_matvec(inp_hbm, conn_hbm, out_hbm, bests_hbm,
               buf0, buf1, obuf, minp, bestv, sem0, sem1):
    wid = lax.axis_index("s") * 2 + lax.axis_index("c")
    pltpu.sync_copy(inp_hbm, minp)

    lane = lax.iota(jnp.int32, 16)
    zero16 = jnp.zeros((16,), jnp.int32)
    bestv[...] = jnp.full((16,), jnp.int32(-2**31 + 1), jnp.int32)

    bufs = (buf0, buf1)
    sems = (sem0, sem1)

    def start(t, buf, sem):
        chunk = wid + t * NW

        @pl.when(chunk < NCHUNK)
        def _():
            pltpu.async_copy(conn_hbm.at[pl.ds(chunk * CH, CH)], buf, sem)

    def process(t, buf, sem):
        chunk = wid + t * NW

        @pl.when(chunk < NCHUNK)
        def _():
            pltpu.make_async_copy(conn_hbm.at[pl.ds(chunk * CH, CH)],
                                  buf, sem).wait()

            def col_body(j, accs):
                colv = (j + lane) & (INPUT_SIZE - 1)
                sv = plsc.load_gather(minp, [zero16, colv])
                new = []
                for g in range(G):
                    v = plsc.load_gather(buf, [g * 16 + lane, colv])
                    new.append(accs[g] + v * sv)
                return tuple(new)

            accs = lax.fori_loop(
                0, INPUT_SIZE, col_body,
                tuple(jnp.zeros((16,), jnp.float32) for _ in range(G)),
                unroll=8)

            best = bestv[...]
            for g in range(G):
                obuf[pl.ds(g * 16, 16)] = accs[g]
                rows = chunk * CH + g * 16 + lane
                key = (accs[g].astype(jnp.int32) << 17) | (131071 - rows)
                best = jnp.maximum(best, key)
            bestv[...] = best
            pltpu.sync_copy(obuf, out_hbm.at[pl.ds(chunk * CH, CH)])

    start(0, buf0, sem0)
    start(1, buf1, sem1)

    def pair_body(i, carry):
        t = 2 * i
        process(t, buf0, sem0)
        start(t + 2, buf0, sem0)
        process(t + 1, buf1, sem1)
        start(t + 3, buf1, sem1)
        return carry

    lax.fori_loop(0, TMAX // 2, pair_body, jnp.int32(0))

    pltpu.sync_copy(bestv, bests_hbm.at[wid])


def kernel(input_array, connected):
    inp = input_array.astype(jnp.float32).reshape(1, INPUT_SIZE)
    overlaps, bests = _sc_matvec(inp, connected)
    winner = 131071 - (jnp.max(bests) & 131071)
    return overlaps, winner
